# speedup vs baseline: 2.9646x; 2.9646x over previous
import functools

import jax
import jax.numpy as jnp
from jax import lax
from jax.experimental import pallas as pl
from jax.experimental.pallas import tpu as pltpu
from jax.experimental.pallas import tpu_sc as plsc

CHUNK = 128


def _make_lookup(num_workers, chunks_per_worker, embed):
    rows_per_worker = chunks_per_worker * CHUNK
    total_rows = num_workers * rows_per_worker

    mesh = plsc.VectorSubcoreMesh(core_axis_name="c", subcore_axis_name="s")

    @functools.partial(
        pl.kernel,
        out_type=jax.ShapeDtypeStruct((total_rows, embed), jnp.float32),
        mesh=mesh,
        scratch_types=[
            pltpu.VMEM((chunks_per_worker, CHUNK), jnp.int32),
            pltpu.VMEM((CHUNK, embed), jnp.float32),
            pltpu.SemaphoreType.DMA,
        ],
    )
    def lookup(x_hbm, table_hbm, out_hbm, idx_v, rows_v, sem):
        num_cores = 2
        wid = lax.axis_index("s") * num_cores + lax.axis_index("c")
        base = wid * rows_per_worker

        pltpu.sync_copy(x_hbm.at[wid], idx_v)

        def body(j, carry):
            pltpu.async_copy(table_hbm.at[idx_v.at[j]], rows_v, sem).wait()
            pltpu.sync_copy(rows_v, out_hbm.at[pl.ds(base + j * CHUNK, CHUNK)])
            return carry

        lax.fori_loop(0, chunks_per_worker, body, 0)

    return lookup


def kernel(x, table):
    batch, hist = x.shape
    vocab, embed = table.shape
    total = batch * hist
    num_workers = 32
    assert total % (num_workers * CHUNK) == 0
    chunks_per_worker = total // (num_workers * CHUNK)

    x_flat = x.reshape(num_workers, chunks_per_worker, CHUNK).astype(jnp.int32)
    lookup = _make_lookup(num_workers, chunks_per_worker, embed)
    out = lookup(x_flat, table)
    return out.reshape(batch, hist, embed)


# speedup vs baseline: 3.3248x; 1.1215x over previous
import functools

import jax
import jax.numpy as jnp
from jax import lax
from jax.experimental import pallas as pl
from jax.experimental.pallas import tpu as pltpu
from jax.experimental.pallas import tpu_sc as plsc

CHUNK = 128


def _make_lookup(num_workers, chunks_per_worker, embed):
    rows_per_worker = chunks_per_worker * CHUNK
    total_rows = num_workers * rows_per_worker

    mesh = plsc.VectorSubcoreMesh(core_axis_name="c", subcore_axis_name="s")

    assert chunks_per_worker % 2 == 0 and chunks_per_worker >= 4

    @functools.partial(
        pl.kernel,
        out_type=jax.ShapeDtypeStruct((total_rows, embed), jnp.float32),
        mesh=mesh,
        scratch_types=[
            pltpu.VMEM((chunks_per_worker, CHUNK), jnp.int32),
            pltpu.VMEM((CHUNK, embed), jnp.float32),
            pltpu.VMEM((CHUNK, embed), jnp.float32),
            pltpu.SemaphoreType.DMA,
            pltpu.SemaphoreType.DMA,
        ],
    )
    def lookup(x_hbm, table_hbm, out_hbm, idx_v, rows_a, rows_b, sem_a, sem_b):
        num_cores = 2
        wid = lax.axis_index("s") * num_cores + lax.axis_index("c")
        base = wid * rows_per_worker

        pltpu.sync_copy(x_hbm.at[wid], idx_v)

        def gather_start(j, buf, sem):
            pltpu.async_copy(table_hbm.at[idx_v.at[j]], buf, sem)

        def gather_wait(j, buf, sem):
            pltpu.make_async_copy(table_hbm.at[idx_v.at[j]], buf, sem).wait()

        def writeback(j, buf):
            pltpu.sync_copy(buf, out_hbm.at[pl.ds(base + j * CHUNK, CHUNK)])

        gather_start(0, rows_a, sem_a)
        gather_start(1, rows_b, sem_b)

        def body(jj, carry):
            j0 = 2 * jj
            gather_wait(j0, rows_a, sem_a)
            writeback(j0, rows_a)
            gather_start(j0 + 2, rows_a, sem_a)
            gather_wait(j0 + 1, rows_b, sem_b)
            writeback(j0 + 1, rows_b)
            gather_start(j0 + 3, rows_b, sem_b)
            return carry

        last = chunks_per_worker - 2
        lax.fori_loop(0, chunks_per_worker // 2 - 1, body, 0)
        gather_wait(last, rows_a, sem_a)
        writeback(last, rows_a)
        gather_wait(last + 1, rows_b, sem_b)
        writeback(last + 1, rows_b)

    return lookup


def kernel(x, table):
    batch, hist = x.shape
    vocab, embed = table.shape
    total = batch * hist
    num_workers = 32
    assert total % (num_workers * CHUNK) == 0
    chunks_per_worker = total // (num_workers * CHUNK)

    x_flat = x.reshape(num_workers, chunks_per_worker, CHUNK).astype(jnp.int32)
    lookup = _make_lookup(num_workers, chunks_per_worker, embed)
    out = lookup(x_flat, table)
    return out.reshape(batch, hist, embed)
